# baseline (speedup 1.0000x reference)
import functools

import jax
import jax.numpy as jnp
from jax import lax
from jax.experimental import pallas as pl
from jax.experimental.pallas import tpu as pltpu
from jax.experimental.pallas import tpu_sc as plsc

_NC, _NS, _L = 2, 16, 16
_NW = _NC * _NS
_GROUP = 32


def _make_sc_kernel(N, D2, MD, MS, n_per_w, n_groups):
    rows_per_w = 2 * n_per_w
    g_rows = 2 * _GROUP
    mesh = plsc.VectorSubcoreMesh(core_axis_name="c", subcore_axis_name="s")

    @functools.partial(
        pl.kernel,
        mesh=mesh,
        out_type=jax.ShapeDtypeStruct((2 * N, D2), jnp.float32),
        scratch_types=[
            pltpu.VMEM((n_per_w,), jnp.int32),
            pltpu.VMEM((n_per_w,), jnp.int32),
            pltpu.VMEM((n_groups * g_rows,), jnp.int32),
            pltpu.VMEM((g_rows, D2), jnp.float32),
            pltpu.SemaphoreType.DMA,
            pltpu.SemaphoreType.DMA,
        ],
    )
    def k(tab_hbm, d_hbm, s_hbm, out_hbm, dv, sv, iv, gbuf, gsem, osem):
        wid = lax.axis_index("s") * _NC + lax.axis_index("c")
        tok0 = wid * n_per_w
        row0 = wid * rows_per_w

        pltpu.sync_copy(d_hbm.at[pl.ds(tok0, n_per_w)], dv)
        pltpu.sync_copy(s_hbm.at[pl.ds(tok0, n_per_w)], sv)

        lane = lax.iota(jnp.int32, _L)
        perm_lo = lane >> 1
        perm_hi = perm_lo + _L // 2
        even = (lane & 1) == 0
        for c in range(n_per_w // _L):
            d = jnp.clip(dv[pl.ds(c * _L, _L)], 0, MD - 1)
            s = jnp.clip(sv[pl.ds(c * _L, _L)], 0, MS - 1) + MD
            lo = jnp.where(even, d.at[perm_lo].get(mode="promise_in_bounds"),
                           s.at[perm_lo].get(mode="promise_in_bounds"))
            hi = jnp.where(even, d.at[perm_hi].get(mode="promise_in_bounds"),
                           s.at[perm_hi].get(mode="promise_in_bounds"))
            iv[pl.ds(2 * c * _L, _L)] = lo
            iv[pl.ds(2 * c * _L + _L, _L)] = hi

        for g in range(n_groups):
            idx = iv.at[pl.ds(g * g_rows, g_rows)]
            pltpu.async_copy(tab_hbm.at[idx], gbuf, gsem).wait()
            pltpu.async_copy(
                gbuf, out_hbm.at[pl.ds(row0 + g * g_rows, g_rows)], osem
            ).wait()

    return k


def kernel(seq_len, tree_depths, sibling_indices, depth_embedding,
           sibling_embedding, pos_embedding):
    B, S = tree_depths.shape
    N = B * S
    MD, D2 = depth_embedding.shape
    MS = sibling_embedding.shape[0]
    n_per_w = N // _NW
    n_groups = n_per_w // _GROUP

    tab = jnp.concatenate([depth_embedding, sibling_embedding], axis=0)
    d_flat = tree_depths.reshape(N)
    s_flat = sibling_indices.reshape(N)

    k = _make_sc_kernel(N, D2, MD, MS, n_per_w, n_groups)
    out = k(tab, d_flat, s_flat)
    return out.reshape(B, S, 2 * D2)


# speedup vs baseline: 1.0039x; 1.0039x over previous
import functools

import jax
import jax.numpy as jnp
from jax import lax
from jax.experimental import pallas as pl
from jax.experimental.pallas import tpu as pltpu
from jax.experimental.pallas import tpu_sc as plsc

_NC, _NS, _L = 2, 16, 16
_NW = _NC * _NS
_GROUP = 32


def _make_sc_kernel(N, D2, MD, MS, n_per_w, n_groups):
    rows_per_w = 2 * n_per_w
    g_rows = 2 * _GROUP
    mesh = plsc.VectorSubcoreMesh(core_axis_name="c", subcore_axis_name="s")

    @functools.partial(
        pl.kernel,
        mesh=mesh,
        out_type=jax.ShapeDtypeStruct((2 * N, D2), jnp.float32),
        scratch_types=[
            pltpu.VMEM((n_per_w,), jnp.int32),
            pltpu.VMEM((n_per_w,), jnp.int32),
            pltpu.VMEM((2 * n_per_w,), jnp.int32),
            pltpu.VMEM((2, g_rows, D2), jnp.float32),
            pltpu.SemaphoreType.DMA,
            pltpu.SemaphoreType.DMA,
        ],
    )
    def k(tab_hbm, d_hbm, s_hbm, out_hbm, dv, sv, iv, gbuf, gsem, osem):
        wid = lax.axis_index("s") * _NC + lax.axis_index("c")
        tok0 = wid * n_per_w
        row0 = wid * rows_per_w

        pltpu.sync_copy(d_hbm.at[pl.ds(tok0, n_per_w)], dv)
        pltpu.sync_copy(s_hbm.at[pl.ds(tok0, n_per_w)], sv)

        lane = lax.iota(jnp.int32, _L)
        perm_lo = lane >> 1
        perm_hi = perm_lo + _L // 2
        even = (lane & 1) == 0
        for c in range(n_per_w // _L):
            d = jnp.clip(dv[pl.ds(c * _L, _L)], 0, MD - 1)
            s = jnp.clip(sv[pl.ds(c * _L, _L)], 0, MS - 1) + MD
            lo = jnp.where(even, d.at[perm_lo].get(mode="promise_in_bounds"),
                           s.at[perm_lo].get(mode="promise_in_bounds"))
            hi = jnp.where(even, d.at[perm_hi].get(mode="promise_in_bounds"),
                           s.at[perm_hi].get(mode="promise_in_bounds"))
            iv[pl.ds(2 * c * _L, _L)] = lo
            iv[pl.ds(2 * c * _L + _L, _L)] = hi

        def gather(g, slot):
            idx = iv.at[pl.ds(g * g_rows, g_rows)]
            return pltpu.async_copy(tab_hbm.at[idx], gbuf.at[slot], gsem)

        def put(g, slot):
            dst = out_hbm.at[pl.ds(row0 + g * g_rows, g_rows)]
            return pltpu.async_copy(gbuf.at[slot], dst, osem)

        gh = {0: gather(0, 0)}
        oh = {}
        for g in range(n_groups):
            gh[g].wait()
            oh[g] = put(g, g & 1)
            if g + 1 < n_groups:
                if g >= 1:
                    oh[g - 1].wait()
                gh[g + 1] = gather(g + 1, (g + 1) & 1)
        oh[n_groups - 2].wait()
        oh[n_groups - 1].wait()

    return k


def kernel(seq_len, tree_depths, sibling_indices, depth_embedding,
           sibling_embedding, pos_embedding):
    B, S = tree_depths.shape
    N = B * S
    MD, D2 = depth_embedding.shape
    MS = sibling_embedding.shape[0]
    n_per_w = N // _NW
    n_groups = n_per_w // _GROUP

    tab = jnp.concatenate([depth_embedding, sibling_embedding], axis=0)
    d_flat = tree_depths.reshape(N)
    s_flat = sibling_indices.reshape(N)

    k = _make_sc_kernel(N, D2, MD, MS, n_per_w, n_groups)
    out = k(tab, d_flat, s_flat)
    return out.reshape(B, S, 2 * D2)


# speedup vs baseline: 3.3187x; 3.3058x over previous
import functools

import jax
import jax.numpy as jnp
from jax import lax
from jax.experimental import pallas as pl
from jax.experimental.pallas import tpu as pltpu
from jax.experimental.pallas import tpu_sc as plsc

_NC, _NS, _L = 2, 16, 16
_NW = _NC * _NS
_GROUP = 32


def _build_fused_table(dep, sib):
    MD, D2 = dep.shape
    MS = sib.shape[0]

    def body(dep_ref, sib_ref, out_ref):
        d = jnp.broadcast_to(dep_ref[...][:, None, :], (MD, MS, D2))
        s = jnp.broadcast_to(sib_ref[...][None, :, :], (MD, MS, D2))
        out_ref[...] = jnp.concatenate([d, s], axis=2).reshape(MD * MS, 2 * D2)

    return pl.pallas_call(
        body,
        out_shape=jax.ShapeDtypeStruct((MD * MS, 2 * D2), jnp.float32),
    )(dep, sib)


def _make_sc_kernel(N, D2, MD, MS, n_per_w, n_groups):
    D = 2 * D2
    mesh = plsc.VectorSubcoreMesh(core_axis_name="c", subcore_axis_name="s")

    @functools.partial(
        pl.kernel,
        mesh=mesh,
        out_type=jax.ShapeDtypeStruct((N, D), jnp.float32),
        scratch_types=[
            pltpu.VMEM((n_per_w,), jnp.int32),
            pltpu.VMEM((n_per_w,), jnp.int32),
            pltpu.VMEM((n_per_w,), jnp.int32),
            pltpu.VMEM((2, _GROUP, D), jnp.float32),
            pltpu.SemaphoreType.DMA,
            pltpu.SemaphoreType.DMA,
        ],
    )
    def k(tab_hbm, d_hbm, s_hbm, out_hbm, dv, sv, iv, gbuf, gsem, osem):
        wid = lax.axis_index("s") * _NC + lax.axis_index("c")
        tok0 = wid * n_per_w

        pltpu.sync_copy(d_hbm.at[pl.ds(tok0, n_per_w)], dv)
        pltpu.sync_copy(s_hbm.at[pl.ds(tok0, n_per_w)], sv)

        for c in range(n_per_w // _L):
            d = jnp.clip(dv[pl.ds(c * _L, _L)], 0, MD - 1)
            s = jnp.clip(sv[pl.ds(c * _L, _L)], 0, MS - 1)
            iv[pl.ds(c * _L, _L)] = d * MS + s

        def gather(g, slot):
            idx = iv.at[pl.ds(g * _GROUP, _GROUP)]
            return pltpu.async_copy(tab_hbm.at[idx], gbuf.at[slot], gsem)

        def put(g, slot):
            off = pl.multiple_of(tok0 + g * _GROUP, 8)
            dst = out_hbm.at[pl.ds(off, _GROUP)]
            return pltpu.async_copy(gbuf.at[slot], dst, osem)

        gh = {0: gather(0, 0)}
        oh = {}
        for g in range(n_groups):
            gh[g].wait()
            oh[g] = put(g, g & 1)
            if g + 1 < n_groups:
                if g >= 1:
                    oh[g - 1].wait()
                gh[g + 1] = gather(g + 1, (g + 1) & 1)
        oh[n_groups - 2].wait()
        oh[n_groups - 1].wait()

    return k


def kernel(seq_len, tree_depths, sibling_indices, depth_embedding,
           sibling_embedding, pos_embedding):
    B, S = tree_depths.shape
    N = B * S
    MD, D2 = depth_embedding.shape
    MS = sibling_embedding.shape[0]
    n_per_w = N // _NW
    n_groups = n_per_w // _GROUP

    tab = _build_fused_table(depth_embedding, sibling_embedding)
    d_flat = tree_depths.reshape(N)
    s_flat = sibling_indices.reshape(N)

    k = _make_sc_kernel(N, D2, MD, MS, n_per_w, n_groups)
    out = k(tab, d_flat, s_flat)
    return out.reshape(B, S, 2 * D2)


# speedup vs baseline: 3.3281x; 1.0029x over previous
import functools

import jax
import jax.numpy as jnp
from jax import lax
from jax.experimental import pallas as pl
from jax.experimental.pallas import tpu as pltpu
from jax.experimental.pallas import tpu_sc as plsc

_NC, _NS, _L = 2, 16, 16
_NW = _NC * _NS
_GROUP = 16
_NBUF = 4


def _build_fused_table(dep, sib):
    MD, D2 = dep.shape
    MS = sib.shape[0]

    def body(dep_ref, sib_ref, out_ref):
        d = jnp.broadcast_to(dep_ref[...][:, None, :], (MD, MS, D2))
        s = jnp.broadcast_to(sib_ref[...][None, :, :], (MD, MS, D2))
        out_ref[...] = jnp.concatenate([d, s], axis=2).reshape(MD * MS, 2 * D2)

    return pl.pallas_call(
        body,
        out_shape=jax.ShapeDtypeStruct((MD * MS, 2 * D2), jnp.float32),
    )(dep, sib)


def _make_sc_kernel(N, D2, MD, MS, n_per_w, n_groups):
    D = 2 * D2
    mesh = plsc.VectorSubcoreMesh(core_axis_name="c", subcore_axis_name="s")

    @functools.partial(
        pl.kernel,
        mesh=mesh,
        out_type=jax.ShapeDtypeStruct((N, D), jnp.float32),
        scratch_types=[
            pltpu.VMEM((n_per_w,), jnp.int32),
            pltpu.VMEM((n_per_w,), jnp.int32),
            pltpu.VMEM((n_per_w,), jnp.int32),
            pltpu.VMEM((_NBUF, _GROUP, D), jnp.float32),
            pltpu.SemaphoreType.DMA,
            pltpu.SemaphoreType.DMA,
        ],
    )
    def k(tab_hbm, d_hbm, s_hbm, out_hbm, dv, sv, iv, gbuf, gsem, osem):
        wid = lax.axis_index("s") * _NC + lax.axis_index("c")
        tok0 = wid * n_per_w

        pltpu.sync_copy(d_hbm.at[pl.ds(tok0, n_per_w)], dv)
        pltpu.sync_copy(s_hbm.at[pl.ds(tok0, n_per_w)], sv)

        for c in range(n_per_w // _L):
            d = jnp.clip(dv[pl.ds(c * _L, _L)], 0, MD - 1)
            s = jnp.clip(sv[pl.ds(c * _L, _L)], 0, MS - 1)
            iv[pl.ds(c * _L, _L)] = d * MS + s

        def gather(g, slot):
            idx = iv.at[pl.ds(g * _GROUP, _GROUP)]
            return pltpu.async_copy(tab_hbm.at[idx], gbuf.at[slot], gsem)

        def put(g, slot):
            off = pl.multiple_of(tok0 + g * _GROUP, 8)
            dst = out_hbm.at[pl.ds(off, _GROUP)]
            return pltpu.async_copy(gbuf.at[slot], dst, osem)

        gh, oh = {}, {}
        unwaited = set()
        for g in range(min(_NBUF - 1, n_groups)):
            gh[g] = gather(g, g % _NBUF)
        for g in range(n_groups):
            gh[g].wait()
            oh[g] = put(g, g % _NBUF)
            unwaited.add(g)
            ng = g + _NBUF - 1
            if ng < n_groups:
                prev = ng - _NBUF
                if prev >= 0:
                    oh[prev].wait()
                    unwaited.discard(prev)
                gh[ng] = gather(ng, ng % _NBUF)
        for g in sorted(unwaited):
            oh[g].wait()

    return k


def kernel(seq_len, tree_depths, sibling_indices, depth_embedding,
           sibling_embedding, pos_embedding):
    B, S = tree_depths.shape
    N = B * S
    MD, D2 = depth_embedding.shape
    MS = sibling_embedding.shape[0]
    n_per_w = N // _NW
    n_groups = n_per_w // _GROUP

    tab = _build_fused_table(depth_embedding, sibling_embedding)
    d_flat = tree_depths.reshape(N)
    s_flat = sibling_indices.reshape(N)

    k = _make_sc_kernel(N, D2, MD, MS, n_per_w, n_groups)
    out = k(tab, d_flat, s_flat)
    return out.reshape(B, S, 2 * D2)
